# baseline (speedup 1.0000x reference)
import functools

import jax
import jax.numpy as jnp
from jax import lax
from jax.experimental import pallas as pl
from jax.experimental.pallas import tpu as pltpu
from jax.experimental.pallas import tpu_sc as plsc

VOCAB = 100000
SHARD = 50000
D = 128
B = 16384 * 50
NC, NS = 2, 16
NW = NC * NS
B_PER_W = B // NW
G = 128
NG = B_PER_W // G

_mesh = plsc.VectorSubcoreMesh(core_axis_name="c", subcore_axis_name="s")


@functools.partial(
    pl.kernel,
    mesh=_mesh,
    out_type=jax.ShapeDtypeStruct((B, D), jnp.float32),
    scratch_types=[
        pltpu.VMEM((G,), jnp.int32),
        pltpu.VMEM((G, D), jnp.float32),
        pltpu.SemaphoreType.DMA,
    ],
)
def _emb_lookup(idx_hbm, tab_hbm, out_hbm, idx_v, rows_v, sem):
    wid = lax.axis_index("s") * NC + lax.axis_index("c")

    def chunk(g, _):
        r = wid * NG + g
        start = pl.multiple_of(r * G, G)
        pltpu.sync_copy(idx_hbm.at[r], idx_v)

        def fix(i, _):
            sl = pl.ds(i * 16, 16)
            idx_v[sl] = jnp.minimum(idx_v[sl], SHARD)
            return 0

        lax.fori_loop(0, G // 16, fix, 0)
        pltpu.async_copy(tab_hbm.at[idx_v], rows_v, sem).wait()
        pltpu.sync_copy(rows_v, out_hbm.at[pl.ds(start, G)])
        return 0

    lax.fori_loop(0, NG, chunk, 0)


def kernel(input, weight):
    idx = input.astype(jnp.int32).reshape(B // G, G)
    tab = jnp.concatenate([weight, jnp.zeros((8, D), jnp.float32)], axis=0)
    out = _emb_lookup(idx, tab)
    return out.reshape(input.shape[0], input.shape[1], D)
